# baseline (speedup 1.0000x reference)
import functools
import math

import jax
import jax.numpy as jnp
from jax import lax
from jax.experimental import pallas as pl
from jax.experimental.pallas import tpu as pltpu
from jax.experimental.pallas import tpu_sc as plsc

_NUM_EXPERTS = 16
_CAPACITY_FACTOR = 1.25
_LANES = 16
_UNROLL = 16


@functools.partial(jax.jit, static_argnums=(3, 4))
def _dispatch(flat_x, ids, wflat, C, G):
    T, D = flat_x.shape
    TK = ids.shape[0]
    K = TK // T
    E = _NUM_EXPERTS
    HALF = C // 2
    HTK = TK // 2
    NCHUNK = HALF // G

    mesh = plsc.VectorSubcoreMesh(core_axis_name="c", subcore_axis_name="s",
                                  num_cores=2, num_subcores=16)

    def body(ids_hbm, x_hbm, w_hbm, out_x, out_w, out_t,
             ids_v, wflat_v, mine_v, listA_v, listB_v, tok0_v, tok1_v,
             rows0_v, rows1_v, wout_v, tout_v, shared_pos,
             sem_in, sem_g0, sem_g1, sem_o0, sem_o1):
        cid = lax.axis_index("c")
        sid = lax.axis_index("s")
        e = cid * (E // 2) + sid // 2
        h = sid % 2
        my_lo = h * HALF

        wcopy = pltpu.async_copy(w_hbm, wflat_v, sem_in)
        pltpu.sync_copy(ids_hbm.at[pl.ds(h * HTK, HTK)], ids_v)

        lanes = lax.broadcasted_iota(jnp.int32, (_LANES,), 0)
        slot0 = h * HTK
        def scan_body(i, cnt_v):
            for u in range(_UNROLL):
                idx = i * _UNROLL + u
                x = ids_v[pl.ds(idx * _LANES, _LANES)]
                m = x == e
                csum = plsc.cumsum(m.astype(jnp.int32))
                dst = cnt_v + csum - 1
                m2 = jnp.logical_and(m, dst < C)
                dst_safe = jnp.clip(dst, 0, C - 1)
                slot = slot0 + idx * _LANES + lanes
                plsc.store_scatter(mine_v, [dst_safe], slot, mask=m2)
                cnt_v = cnt_v + plsc.all_reduce_population_count(m)
            return cnt_v

        cnt_v = lax.fori_loop(0, HTK // _LANES // _UNROLL, scan_body,
                              jnp.zeros((_LANES,), jnp.int32))

        mine_v[pl.ds(C, _LANES)] = cnt_v
        pltpu.sync_copy(mine_v, shared_pos.at[sid])
        plsc.subcore_barrier()
        pair0 = (sid // 2) * 2
        pltpu.sync_copy(shared_pos.at[pair0], listA_v)
        pltpu.sync_copy(shared_pos.at[pair0 + 1], listB_v)
        nA = jnp.max(listA_v[pl.ds(C, _LANES)])
        nB = jnp.max(listB_v[pl.ds(C, _LANES)])
        cnt_c = jnp.minimum(nA + nB, C)
        g_lo = my_lo
        g_len = HALF
        nval = jnp.clip(cnt_c - g_lo, 0, g_len)

        def combined(base):
            cl = base + lanes
            inA = cl < nA
            iA = jnp.clip(cl, 0, C - 1)
            iB = jnp.clip(cl - nA, 0, C - 1)
            pA = plsc.load_gather(listA_v, [iA])
            pB = plsc.load_gather(listB_v, [iB])
            return jnp.where(inA, pA, pB), cl < cnt_c

        n_gather = (nval + G - 1) // G
        nch = g_len // G

        def build_tok(j, tok_b):
            base = g_lo + j * G
            for t in range(G // _LANES):
                cl = base + t * _LANES + lanes
                p, valid = combined(base + t * _LANES)
                psafe = jnp.where(valid, p, cl * K)
                tok_b[pl.ds(t * _LANES, _LANES)] = psafe // K

        @pl.when(n_gather >= 1)
        def _():
            build_tok(0, tok0_v)
            pltpu.async_copy(x_hbm.at[tok0_v], rows0_v, sem_g0)

        wcopy.wait()

        def wt_body(t, _):
            base = my_lo + t * _LANES
            p, valid = combined(base)
            psafe = jnp.where(valid, p, 0)
            w = plsc.load_gather(wflat_v, [psafe])
            wout_v[pl.ds(t * _LANES, _LANES)] = jnp.where(valid, w, 0.0)
            tout_v[pl.ds(t * _LANES, _LANES)] = jnp.where(valid, psafe // K, -1)
            return 0

        lax.fori_loop(0, HALF // _LANES, wt_body, 0)
        pltpu.sync_copy(wout_v, out_w.at[e, pl.ds(my_lo, HALF)])
        pltpu.sync_copy(tout_v, out_t.at[e, pl.ds(my_lo, HALF)])

        def zero_rows(rows_b, r_lo, r_hi):
            zeros16 = jnp.zeros((_LANES,), jnp.float32)

            def z_body(r, _):
                for v in range(D // _LANES):
                    rows_b[r, pl.ds(v * _LANES, _LANES)] = zeros16
                return 0

            lax.fori_loop(r_lo, r_hi, z_body, 0)

        def step(j, cur, nxt):
            rows_c, tok_c, sg_c, so_c = cur
            rows_n, tok_n, sg_n, so_n = nxt
            pltpu.make_async_copy(x_hbm.at[tok_c], rows_c, sg_c).wait()
            vrows = jnp.clip(nval - j * G, 0, G)

            @pl.when(vrows < G)
            def _():
                zero_rows(rows_c, vrows, G)

            pltpu.async_copy(rows_c, out_x.at[e, pl.ds(g_lo + j * G, G)], so_c)

            @pl.when(j + 1 < n_gather)
            def _():
                build_tok(j + 1, tok_n)

                @pl.when(j >= 1)
                def _():
                    pltpu.make_async_copy(
                        rows_n, out_x.at[e, pl.ds(my_lo, G)], so_n).wait()

                pltpu.async_copy(x_hbm.at[tok_n], rows_n, sg_n)

        buf0 = (rows0_v, tok0_v, sem_g0, sem_o0)
        buf1 = (rows1_v, tok1_v, sem_g1, sem_o1)

        def g_body(j, _):
            @pl.when(j % 2 == 0)
            def _():
                step(j, buf0, buf1)

            @pl.when(j % 2 == 1)
            def _():
                step(j, buf1, buf0)

            return 0

        lax.fori_loop(0, n_gather, g_body, 0)

        @pl.when(n_gather % 2 == 1)
        def _():
            pltpu.make_async_copy(
                rows0_v, out_x.at[e, pl.ds(my_lo, G)], sem_o0).wait()

            @pl.when(n_gather >= 2)
            def _():
                pltpu.make_async_copy(
                    rows1_v, out_x.at[e, pl.ds(my_lo, G)], sem_o1).wait()

        @pl.when(jnp.logical_and(n_gather % 2 == 0, n_gather >= 1))
        def _():
            pltpu.make_async_copy(
                rows1_v, out_x.at[e, pl.ds(my_lo, G)], sem_o1).wait()
            pltpu.make_async_copy(
                rows0_v, out_x.at[e, pl.ds(my_lo, G)], sem_o0).wait()

        @pl.when(n_gather < nch)
        def _():
            zero_rows(rows0_v, 0, G)
            zero_rows(rows1_v, 0, G)

            def out_body(j, _):
                @pl.when(j % 2 == 0)
                def _():
                    @pl.when(j >= n_gather + 2)
                    def _():
                        pltpu.make_async_copy(
                            rows0_v, out_x.at[e, pl.ds(g_lo, G)], sem_o0).wait()

                    pltpu.async_copy(
                        rows0_v, out_x.at[e, pl.ds(g_lo + j * G, G)], sem_o0)

                @pl.when(j % 2 == 1)
                def _():
                    @pl.when(j >= n_gather + 2)
                    def _():
                        pltpu.make_async_copy(
                            rows1_v, out_x.at[e, pl.ds(g_lo, G)], sem_o1).wait()

                    pltpu.async_copy(
                        rows1_v, out_x.at[e, pl.ds(g_lo + j * G, G)], sem_o1)

                return 0

            lax.fori_loop(n_gather, nch, out_body, 0)

            def drain_last(idx):
                @pl.when(idx % 2 == 0)
                def _():
                    pltpu.make_async_copy(
                        rows0_v, out_x.at[e, pl.ds(g_lo, G)], sem_o0).wait()

                @pl.when(idx % 2 == 1)
                def _():
                    pltpu.make_async_copy(
                        rows1_v, out_x.at[e, pl.ds(g_lo, G)], sem_o1).wait()

            drain_last(nch - 1)

            @pl.when(nch - n_gather >= 2)
            def _():
                drain_last(nch - 2)

    run = pl.kernel(
        body,
        out_type=[
            jax.ShapeDtypeStruct((E, C, D), jnp.float32),
            jax.ShapeDtypeStruct((E, C), jnp.float32),
            jax.ShapeDtypeStruct((E, C), jnp.int32),
        ],
        mesh=mesh,
        compiler_params=pltpu.CompilerParams(needs_layout_passes=False),
        scratch_types=[
            pltpu.VMEM((HTK,), jnp.int32),
            pltpu.VMEM((TK,), jnp.float32),
            pltpu.VMEM((C + _LANES,), jnp.int32),
            pltpu.VMEM((C + _LANES,), jnp.int32),
            pltpu.VMEM((C + _LANES,), jnp.int32),
            pltpu.VMEM((G,), jnp.int32),
            pltpu.VMEM((G,), jnp.int32),
            pltpu.VMEM((G, D), jnp.float32),
            pltpu.VMEM((G, D), jnp.float32),
            pltpu.VMEM((HALF,), jnp.float32),
            pltpu.VMEM((HALF,), jnp.int32),
            pltpu.VMEM_SHARED((16, C + _LANES), jnp.int32),
            pltpu.SemaphoreType.DMA,
            pltpu.SemaphoreType.DMA,
            pltpu.SemaphoreType.DMA,
            pltpu.SemaphoreType.DMA,
            pltpu.SemaphoreType.DMA,
        ],
    )
    return tuple(run(ids, flat_x, wflat))


def kernel(inputs, top_k_indices, top_k_weights):
    B, S, D = inputs.shape
    K = top_k_indices.shape[-1]
    T = B * S
    C = int(math.ceil(T * K / _NUM_EXPERTS * _CAPACITY_FACTOR))
    flat_x = inputs.reshape(T, D)
    ids = top_k_indices.reshape(T * K).astype(jnp.int32)
    wflat = top_k_weights.reshape(T * K)
    return _dispatch(flat_x, ids, wflat, C, 32)


# speedup vs baseline: 1.0029x; 1.0029x over previous
import functools
import math

import jax
import jax.numpy as jnp
from jax import lax
from jax.experimental import pallas as pl
from jax.experimental.pallas import tpu as pltpu
from jax.experimental.pallas import tpu_sc as plsc

_NUM_EXPERTS = 16
_CAPACITY_FACTOR = 1.25
_LANES = 16
_UNROLL = 8


@functools.partial(jax.jit, static_argnums=(3, 4))
def _dispatch(flat_x, ids, wflat, C, G):
    T, D = flat_x.shape
    TK = ids.shape[0]
    K = TK // T
    E = _NUM_EXPERTS
    HALF = C // 2
    HTK = TK // 2
    NCHUNK = HALF // G

    mesh = plsc.VectorSubcoreMesh(core_axis_name="c", subcore_axis_name="s",
                                  num_cores=2, num_subcores=16)

    def body(ids_hbm, x_hbm, w_hbm, out_x, out_w, out_t,
             ids_v, wflat_v, mine_v, listA_v, listB_v, tok0_v, tok1_v,
             rows0_v, rows1_v, wout_v, tout_v, shared_pos,
             sem_in, sem_g0, sem_g1, sem_o0, sem_o1):
        cid = lax.axis_index("c")
        sid = lax.axis_index("s")
        e = cid * (E // 2) + sid // 2
        h = sid % 2
        my_lo = h * HALF

        wcopy = pltpu.async_copy(w_hbm, wflat_v, sem_in)
        pltpu.sync_copy(ids_hbm.at[pl.ds(h * HTK, HTK)], ids_v)

        lanes = lax.broadcasted_iota(jnp.int32, (_LANES,), 0)
        slot0 = h * HTK
        def scan_body(i, cnt_v):
            for u in range(_UNROLL):
                idx = i * _UNROLL + u
                x = ids_v[pl.ds(idx * _LANES, _LANES)]
                m = x == e
                csum = plsc.cumsum(m.astype(jnp.int32))
                dst = cnt_v + csum - 1
                m2 = jnp.logical_and(m, dst < C)
                dst_safe = jnp.clip(dst, 0, C - 1)
                slot = slot0 + idx * _LANES + lanes
                plsc.store_scatter(mine_v, [dst_safe], slot, mask=m2)
                cnt_v = cnt_v + plsc.all_reduce_population_count(m)
            return cnt_v

        cnt_v = lax.fori_loop(0, HTK // _LANES // _UNROLL, scan_body,
                              jnp.zeros((_LANES,), jnp.int32))

        mine_v[pl.ds(C, _LANES)] = cnt_v
        pltpu.sync_copy(mine_v, shared_pos.at[sid])
        plsc.subcore_barrier()
        pair0 = (sid // 2) * 2
        pltpu.sync_copy(shared_pos.at[pair0], listA_v)
        pltpu.sync_copy(shared_pos.at[pair0 + 1], listB_v)
        nA = jnp.max(listA_v[pl.ds(C, _LANES)])
        nB = jnp.max(listB_v[pl.ds(C, _LANES)])
        cnt_c = jnp.minimum(nA + nB, C)
        g_lo = my_lo
        g_len = HALF
        nval = jnp.clip(cnt_c - g_lo, 0, g_len)

        def combined(base):
            cl = base + lanes
            inA = cl < nA
            iA = jnp.clip(cl, 0, C - 1)
            iB = jnp.clip(cl - nA, 0, C - 1)
            pA = plsc.load_gather(listA_v, [iA])
            pB = plsc.load_gather(listB_v, [iB])
            return jnp.where(inA, pA, pB), cl < cnt_c

        n_gather = (nval + G - 1) // G
        nch = g_len // G

        def build_tok(j, tok_b):
            base = g_lo + j * G
            for t in range(G // _LANES):
                cl = base + t * _LANES + lanes
                p, valid = combined(base + t * _LANES)
                psafe = jnp.where(valid, p, cl * K)
                tok_b[pl.ds(t * _LANES, _LANES)] = psafe // K

        @pl.when(n_gather >= 1)
        def _():
            build_tok(0, tok0_v)
            pltpu.async_copy(x_hbm.at[tok0_v], rows0_v, sem_g0)

        wcopy.wait()

        def wt_body(t, _):
            base = my_lo + t * _LANES
            p, valid = combined(base)
            psafe = jnp.where(valid, p, 0)
            w = plsc.load_gather(wflat_v, [psafe])
            wout_v[pl.ds(t * _LANES, _LANES)] = jnp.where(valid, w, 0.0)
            tout_v[pl.ds(t * _LANES, _LANES)] = jnp.where(valid, psafe // K, -1)
            return 0

        lax.fori_loop(0, HALF // _LANES, wt_body, 0)
        pltpu.sync_copy(wout_v, out_w.at[e, pl.ds(my_lo, HALF)])
        pltpu.sync_copy(tout_v, out_t.at[e, pl.ds(my_lo, HALF)])

        def zero_rows(rows_b, r_lo, r_hi):
            zeros16 = jnp.zeros((_LANES,), jnp.float32)

            def z_body(r, _):
                for v in range(D // _LANES):
                    rows_b[r, pl.ds(v * _LANES, _LANES)] = zeros16
                return 0

            lax.fori_loop(r_lo, r_hi, z_body, 0)

        def step(j, cur, nxt):
            rows_c, tok_c, sg_c, so_c = cur
            rows_n, tok_n, sg_n, so_n = nxt
            pltpu.make_async_copy(x_hbm.at[tok_c], rows_c, sg_c).wait()
            vrows = jnp.clip(nval - j * G, 0, G)

            @pl.when(vrows < G)
            def _():
                zero_rows(rows_c, vrows, G)

            pltpu.async_copy(rows_c, out_x.at[e, pl.ds(g_lo + j * G, G)], so_c)

            @pl.when(j + 1 < n_gather)
            def _():
                build_tok(j + 1, tok_n)

                @pl.when(j >= 1)
                def _():
                    pltpu.make_async_copy(
                        rows_n, out_x.at[e, pl.ds(my_lo, G)], so_n).wait()

                pltpu.async_copy(x_hbm.at[tok_n], rows_n, sg_n)

        buf0 = (rows0_v, tok0_v, sem_g0, sem_o0)
        buf1 = (rows1_v, tok1_v, sem_g1, sem_o1)

        def g_body(j, _):
            @pl.when(j % 2 == 0)
            def _():
                step(j, buf0, buf1)

            @pl.when(j % 2 == 1)
            def _():
                step(j, buf1, buf0)

            return 0

        lax.fori_loop(0, n_gather, g_body, 0)

        @pl.when(n_gather % 2 == 1)
        def _():
            pltpu.make_async_copy(
                rows0_v, out_x.at[e, pl.ds(my_lo, G)], sem_o0).wait()

            @pl.when(n_gather >= 2)
            def _():
                pltpu.make_async_copy(
                    rows1_v, out_x.at[e, pl.ds(my_lo, G)], sem_o1).wait()

        @pl.when(jnp.logical_and(n_gather % 2 == 0, n_gather >= 1))
        def _():
            pltpu.make_async_copy(
                rows1_v, out_x.at[e, pl.ds(my_lo, G)], sem_o1).wait()
            pltpu.make_async_copy(
                rows0_v, out_x.at[e, pl.ds(my_lo, G)], sem_o0).wait()

        @pl.when(n_gather < nch)
        def _():
            zero_rows(rows0_v, 0, G)
            zero_rows(rows1_v, 0, G)

            def out_body(j, _):
                @pl.when(j % 2 == 0)
                def _():
                    @pl.when(j >= n_gather + 2)
                    def _():
                        pltpu.make_async_copy(
                            rows0_v, out_x.at[e, pl.ds(g_lo, G)], sem_o0).wait()

                    pltpu.async_copy(
                        rows0_v, out_x.at[e, pl.ds(g_lo + j * G, G)], sem_o0)

                @pl.when(j % 2 == 1)
                def _():
                    @pl.when(j >= n_gather + 2)
                    def _():
                        pltpu.make_async_copy(
                            rows1_v, out_x.at[e, pl.ds(g_lo, G)], sem_o1).wait()

                    pltpu.async_copy(
                        rows1_v, out_x.at[e, pl.ds(g_lo + j * G, G)], sem_o1)

                return 0

            lax.fori_loop(n_gather, nch, out_body, 0)

            def drain_last(idx):
                @pl.when(idx % 2 == 0)
                def _():
                    pltpu.make_async_copy(
                        rows0_v, out_x.at[e, pl.ds(g_lo, G)], sem_o0).wait()

                @pl.when(idx % 2 == 1)
                def _():
                    pltpu.make_async_copy(
                        rows1_v, out_x.at[e, pl.ds(g_lo, G)], sem_o1).wait()

            drain_last(nch - 1)

            @pl.when(nch - n_gather >= 2)
            def _():
                drain_last(nch - 2)

    run = pl.kernel(
        body,
        out_type=[
            jax.ShapeDtypeStruct((E, C, D), jnp.float32),
            jax.ShapeDtypeStruct((E, C), jnp.float32),
            jax.ShapeDtypeStruct((E, C), jnp.int32),
        ],
        mesh=mesh,
        compiler_params=pltpu.CompilerParams(needs_layout_passes=False),
        scratch_types=[
            pltpu.VMEM((HTK,), jnp.int32),
            pltpu.VMEM((TK,), jnp.float32),
            pltpu.VMEM((C + _LANES,), jnp.int32),
            pltpu.VMEM((C + _LANES,), jnp.int32),
            pltpu.VMEM((C + _LANES,), jnp.int32),
            pltpu.VMEM((G,), jnp.int32),
            pltpu.VMEM((G,), jnp.int32),
            pltpu.VMEM((G, D), jnp.float32),
            pltpu.VMEM((G, D), jnp.float32),
            pltpu.VMEM((HALF,), jnp.float32),
            pltpu.VMEM((HALF,), jnp.int32),
            pltpu.VMEM_SHARED((16, C + _LANES), jnp.int32),
            pltpu.SemaphoreType.DMA,
            pltpu.SemaphoreType.DMA,
            pltpu.SemaphoreType.DMA,
            pltpu.SemaphoreType.DMA,
            pltpu.SemaphoreType.DMA,
        ],
    )
    return tuple(run(ids, flat_x, wflat))


def kernel(inputs, top_k_indices, top_k_weights):
    B, S, D = inputs.shape
    K = top_k_indices.shape[-1]
    T = B * S
    C = int(math.ceil(T * K / _NUM_EXPERTS * _CAPACITY_FACTOR))
    flat_x = inputs.reshape(T, D)
    ids = top_k_indices.reshape(T * K).astype(jnp.int32)
    wflat = top_k_weights.reshape(T * K)
    return _dispatch(flat_x, ids, wflat, C, 32)
